# baseline (speedup 1.0000x reference)
import functools

import jax
import jax.numpy as jnp
from jax import lax
from jax.experimental import pallas as pl
from jax.experimental.pallas import tpu as pltpu
from jax.experimental.pallas import tpu_sc as plsc

_N = 1_000_000
_NW = 32
_UNROLL = 4

_BLK = 65536
_NSC = 8 * _BLK
_TAILB = (_N // 128) * 128

_STRIDE = _NSC // _NW
_CHUNK = 16384
_ITERS = _CHUNK // (16 * _UNROLL)
_ITERS_T = _ITERS + 1

_OFF = 0x3F3504F3
_LN2 = 0.6931471805599453
_PC = (0.1193119419053133, -0.18680964217965043, 0.2049179463920517,
       -0.24908270227751894, 0.33314670851721606, -0.5000114538020157,
       1.000000964626097)


def _neg_ln(x):
    bits = lax.bitcast_convert_type(x, jnp.int32)
    e = (bits - _OFF) >> 23
    m = lax.bitcast_convert_type(bits - (e << 23), jnp.float32)
    f = m - jnp.float32(1.0)
    poly = jnp.full((16,), _PC[0], jnp.float32)
    for c in _PC[1:]:
        poly = poly * f + jnp.float32(c)
    return jnp.float32(0.0) - (f * poly + e.astype(jnp.float32) * jnp.float32(_LN2))


_mesh = plsc.VectorSubcoreMesh(core_axis_name="c", subcore_axis_name="s")


@functools.partial(
    pl.kernel,
    mesh=_mesh,
    out_type=(jax.ShapeDtypeStruct((_NW * 16,), jnp.float32),
              jax.ShapeDtypeStruct((_NW * 16,), jnp.int32)),
    scratch_types=(pltpu.VMEM((_CHUNK + 64,), jnp.float32),
                   pltpu.VMEM((_CHUNK + 64,), jnp.float32),
                   pltpu.VMEM((16,), jnp.float32),
                   pltpu.VMEM((16,), jnp.int32),
                   pltpu.SemaphoreType.DMA,
                   pltpu.SemaphoreType.DMA,
                   pltpu.SemaphoreType.DMA,
                   pltpu.SemaphoreType.DMA),
)
def _sc_scan(p_hbm, u_hbm, vals_out, idx_out, p_v, u_v, rv, ri,
             sem_p, sem_u, sem_pt, sem_ut):
    w = lax.axis_index("s") * 2 + lax.axis_index("c")
    b = jnp.minimum(w * _STRIDE, _NSC - _CHUNK)
    b = pl.multiple_of(b, 16)
    cp_p = pltpu.async_copy(p_hbm.at[pl.ds(b, _CHUNK)], p_v.at[pl.ds(0, _CHUNK)], sem_p)
    cp_u = pltpu.async_copy(u_hbm.at[pl.ds(b, _CHUNK)], u_v.at[pl.ds(0, _CHUNK)], sem_u)
    cp_pt = pltpu.async_copy(p_hbm.at[pl.ds(_TAILB, 64)], p_v.at[pl.ds(_CHUNK, 64)], sem_pt)
    cp_ut = pltpu.async_copy(u_hbm.at[pl.ds(_TAILB, 64)], u_v.at[pl.ds(_CHUNK, 64)], sem_ut)
    cp_p.wait()
    cp_u.wait()
    cp_pt.wait()
    cp_ut.wait()

    def body(i, carry):
        bv, bc = carry
        base_code = i * _UNROLL
        v = []
        for j in range(_UNROLL):
            off = base_code + j
            pv = p_v[pl.ds(off * 16, 16)]
            uv = u_v[pl.ds(off * 16, 16)]
            v.append(_neg_ln(uv) * jnp.exp(jnp.float32(0.0) - pv))
        lt01 = v[1] < v[0]
        va = jnp.where(lt01, v[1], v[0])
        ca = jnp.where(lt01, base_code + 1, base_code)
        lt23 = v[3] < v[2]
        vb = jnp.where(lt23, v[3], v[2])
        cb = jnp.where(lt23, base_code + 3, base_code + 2)
        ltab = vb < va
        vw = jnp.where(ltab, vb, va)
        cw = jnp.where(ltab, cb, ca)
        lt = vw < bv
        return jnp.where(lt, vw, bv), jnp.where(lt, cw, bc)

    init = (jnp.full((16,), jnp.inf, jnp.float32), jnp.zeros((16,), jnp.int32))
    bv, bc = lax.fori_loop(0, _ITERS_T, body, init)
    in_tail = bc >= _ITERS * _UNROLL
    base = jnp.where(in_tail, _TAILB - _ITERS * _UNROLL * 16, b)
    rv[...] = bv
    ri[...] = base + bc * 16 + lax.iota(jnp.int32, 16)
    o = pl.multiple_of(w * 16, 16)
    pltpu.sync_copy(rv, vals_out.at[pl.ds(o, 16)])
    pltpu.sync_copy(ri, idx_out.at[pl.ds(o, 16)])


_TCFULL = (_TAILB - _NSC) // _BLK
_TCPAIRS = _TCFULL // 2
_TAIL = _TAILB - _NSC - _TCFULL * _BLK


def _tc_chunk_min(v, base, rows):
    gidx = (lax.broadcasted_iota(jnp.int32, (rows, 128), 0) * 128
            + lax.broadcasted_iota(jnp.int32, (rows, 128), 1) + base)
    m = jnp.min(v)
    ci = jnp.min(jnp.where(v == m, gidx, jnp.int32(2**31 - 1)))
    return m, ci


def _tc_scan_body(p_hbm, u_hbm, val_out, idx_out, pb0, ub0, pb1, ub1, sems):
    def start(g, pb, ub, si):
        src = pl.ds(pl.multiple_of(_NSC + g * _BLK, 128), _BLK)
        pltpu.make_async_copy(p_hbm.at[src], pb, sems.at[si]).start()
        pltpu.make_async_copy(u_hbm.at[src], ub, sems.at[si + 1]).start()

    def wait(g, pb, ub, si):
        src = pl.ds(pl.multiple_of(_NSC + g * _BLK, 128), _BLK)
        pltpu.make_async_copy(p_hbm.at[src], pb, sems.at[si]).wait()
        pltpu.make_async_copy(u_hbm.at[src], ub, sems.at[si + 1]).wait()

    _SUB = _BLK // 1024

    def chunk(g, pb, ub, carry):
        bv8, bc8 = carry
        pv = pb[...].reshape(_BLK // 128, 128)
        uv = ub[...].reshape(_BLK // 128, 128)
        v = (jnp.float32(0.0) - jnp.log(uv)) * jnp.exp(jnp.float32(0.0) - pv)
        for s in range(_SUB):
            sub = v[s * 8:(s + 1) * 8, :]
            code = g * _SUB + s
            lt = sub < bv8
            bv8 = jnp.where(lt, sub, bv8)
            bc8 = jnp.where(lt, code, bc8)
        return bv8, bc8

    start(0, pb0, ub0, 0)

    def body(i, carry):
        g = i * 2
        wait(g, pb0, ub0, 0)
        start(g + 1, pb1, ub1, 2)
        carry = chunk(g, pb0, ub0, carry)
        wait(g + 1, pb1, ub1, 2)

        @pl.when(g + 2 < _TCFULL)
        def _():
            start(g + 2, pb0, ub0, 0)

        return chunk(g + 1, pb1, ub1, carry)

    init = (jnp.full((8, 128), jnp.inf, jnp.float32),
            jnp.zeros((8, 128), jnp.int32))
    carry = lax.fori_loop(0, _TCPAIRS, body, init)
    if _TCFULL % 2 == 1:
        g = _TCFULL - 1
        wait(g, pb0, ub0, 0)
        carry = chunk(g, pb0, ub0, carry)
    bv8, bc8 = carry

    gidx8 = (_NSC + bc8 * 1024
             + lax.broadcasted_iota(jnp.int32, (8, 128), 0) * 128
             + lax.broadcasted_iota(jnp.int32, (8, 128), 1))
    bv = jnp.min(bv8)
    bi = jnp.min(jnp.where(bv8 == bv, gidx8, jnp.int32(2**31 - 1)))

    tsrc = pl.ds(_NSC + _TCFULL * _BLK, _TAIL)
    pltpu.make_async_copy(p_hbm.at[tsrc], pb0.at[pl.ds(0, _TAIL)], sems.at[0]).start()
    pltpu.make_async_copy(u_hbm.at[tsrc], ub0.at[pl.ds(0, _TAIL)], sems.at[1]).start()
    pltpu.make_async_copy(p_hbm.at[tsrc], pb0.at[pl.ds(0, _TAIL)], sems.at[0]).wait()
    pltpu.make_async_copy(u_hbm.at[tsrc], ub0.at[pl.ds(0, _TAIL)], sems.at[1]).wait()
    pv = pb0[pl.ds(0, _TAIL)].reshape(_TAIL // 128, 128)
    uv = ub0[pl.ds(0, _TAIL)].reshape(_TAIL // 128, 128)
    v = (jnp.float32(0.0) - jnp.log(uv)) * jnp.exp(jnp.float32(0.0) - pv)
    m, ci = _tc_chunk_min(v, _NSC + _TCFULL * _BLK, _TAIL // 128)
    better = (m < bv) | ((m == bv) & (ci < bi))
    val_out[0, 0] = jnp.where(better, m, bv)
    idx_out[0, 0] = jnp.where(better, ci, bi)


_tc_scan = pl.pallas_call(
    _tc_scan_body,
    in_specs=[pl.BlockSpec(memory_space=pl.ANY),
              pl.BlockSpec(memory_space=pl.ANY)],
    out_specs=[pl.BlockSpec(memory_space=pltpu.SMEM),
               pl.BlockSpec(memory_space=pltpu.SMEM)],
    out_shape=[jax.ShapeDtypeStruct((1, 1), jnp.float32),
               jax.ShapeDtypeStruct((1, 1), jnp.int32)],
    scratch_shapes=[pltpu.VMEM((_BLK,), jnp.float32),
                    pltpu.VMEM((_BLK,), jnp.float32),
                    pltpu.VMEM((_BLK,), jnp.float32),
                    pltpu.VMEM((_BLK,), jnp.float32),
                    pltpu.SemaphoreType.DMA((4,))],
)


def _merge_body(v_ref, i_ref, tv_ref, ti_ref, o_ref):
    v = v_ref[...].reshape(4, 128)
    ix = i_ref[...].reshape(4, 128)
    m = jnp.min(v)
    i_sc = jnp.min(jnp.where(v == m, ix, jnp.int32(2**31 - 1)))
    tv = tv_ref[0, 0]
    ti = ti_ref[0, 0]
    tc_wins = (tv < m) | ((tv == m) & (ti < i_sc))
    o_ref[0, 0] = jnp.where(tc_wins, ti, i_sc)


_merge = pl.pallas_call(
    _merge_body,
    in_specs=[pl.BlockSpec(memory_space=pltpu.VMEM),
              pl.BlockSpec(memory_space=pltpu.VMEM),
              pl.BlockSpec(memory_space=pltpu.SMEM),
              pl.BlockSpec(memory_space=pltpu.SMEM)],
    out_specs=pl.BlockSpec(memory_space=pltpu.SMEM),
    out_shape=jax.ShapeDtypeStruct((1, 1), jnp.int32),
)


def kernel(p, u):
    vals, idx = _sc_scan(p, u)
    tv, ti = _tc_scan(p, u)
    out = _merge(vals, idx, tv, ti)
    return out[0, 0]


# speedup vs baseline: 1.0186x; 1.0186x over previous
import functools

import jax
import jax.numpy as jnp
from jax import lax
from jax.experimental import pallas as pl
from jax.experimental.pallas import tpu as pltpu
from jax.experimental.pallas import tpu_sc as plsc

_N = 1_000_000
_NW = 32
_UNROLL = 4

_BLK = 65536
_NSC = 7 * _BLK
_TAILB = (_N // 128) * 128

_STRIDE = _NSC // _NW
_CHUNK = 14336
_HALF = _CHUNK // 2
_ITERS = _CHUNK // (16 * _UNROLL)
_HITERS = _ITERS // 2
_ITERS_T = _ITERS + 1

_OFF = 0x3F3504F3
_LN2 = 0.6931471805599453
_PC = (0.1193119419053133, -0.18680964217965043, 0.2049179463920517,
       -0.24908270227751894, 0.33314670851721606, -0.5000114538020157,
       1.000000964626097)


def _neg_ln(x):
    bits = lax.bitcast_convert_type(x, jnp.int32)
    e = (bits - _OFF) >> 23
    m = lax.bitcast_convert_type(bits - (e << 23), jnp.float32)
    f = m - jnp.float32(1.0)
    poly = jnp.full((16,), _PC[0], jnp.float32)
    for c in _PC[1:]:
        poly = poly * f + jnp.float32(c)
    return jnp.float32(0.0) - (f * poly + e.astype(jnp.float32) * jnp.float32(_LN2))


_mesh = plsc.VectorSubcoreMesh(core_axis_name="c", subcore_axis_name="s")


@functools.partial(
    pl.kernel,
    mesh=_mesh,
    out_type=(jax.ShapeDtypeStruct((_NW * 16,), jnp.float32),
              jax.ShapeDtypeStruct((_NW * 16,), jnp.int32)),
    scratch_types=(pltpu.VMEM((_CHUNK + 64,), jnp.float32),
                   pltpu.VMEM((_CHUNK + 64,), jnp.float32),
                   pltpu.VMEM((16,), jnp.float32),
                   pltpu.VMEM((16,), jnp.int32),
                   pltpu.SemaphoreType.DMA,
                   pltpu.SemaphoreType.DMA,
                   pltpu.SemaphoreType.DMA,
                   pltpu.SemaphoreType.DMA,
                   pltpu.SemaphoreType.DMA,
                   pltpu.SemaphoreType.DMA),
)
def _sc_scan(p_hbm, u_hbm, vals_out, idx_out, p_v, u_v, rv, ri,
             sem_p0, sem_u0, sem_p1, sem_u1, sem_pt, sem_ut):
    w = lax.axis_index("s") * 2 + lax.axis_index("c")
    b = jnp.minimum(w * _STRIDE, _NSC - _CHUNK)
    b = pl.multiple_of(b, 16)
    cp_p0 = pltpu.async_copy(p_hbm.at[pl.ds(b, _HALF)], p_v.at[pl.ds(0, _HALF)], sem_p0)
    cp_u0 = pltpu.async_copy(u_hbm.at[pl.ds(b, _HALF)], u_v.at[pl.ds(0, _HALF)], sem_u0)
    b1 = pl.multiple_of(b + _HALF, 16)
    cp_p1 = pltpu.async_copy(p_hbm.at[pl.ds(b1, _HALF)], p_v.at[pl.ds(_HALF, _HALF)], sem_p1)
    cp_u1 = pltpu.async_copy(u_hbm.at[pl.ds(b1, _HALF)], u_v.at[pl.ds(_HALF, _HALF)], sem_u1)
    cp_pt = pltpu.async_copy(p_hbm.at[pl.ds(_TAILB, 64)], p_v.at[pl.ds(_CHUNK, 64)], sem_pt)
    cp_ut = pltpu.async_copy(u_hbm.at[pl.ds(_TAILB, 64)], u_v.at[pl.ds(_CHUNK, 64)], sem_ut)
    cp_p0.wait()
    cp_u0.wait()

    def body(i, carry):
        bv, bc = carry
        base_code = i * _UNROLL
        v = []
        for j in range(_UNROLL):
            off = base_code + j
            pv = p_v[pl.ds(off * 16, 16)]
            uv = u_v[pl.ds(off * 16, 16)]
            v.append(_neg_ln(uv) * jnp.exp(jnp.float32(0.0) - pv))
        lt01 = v[1] < v[0]
        va = jnp.where(lt01, v[1], v[0])
        ca = jnp.where(lt01, base_code + 1, base_code)
        lt23 = v[3] < v[2]
        vb = jnp.where(lt23, v[3], v[2])
        cb = jnp.where(lt23, base_code + 3, base_code + 2)
        ltab = vb < va
        vw = jnp.where(ltab, vb, va)
        cw = jnp.where(ltab, cb, ca)
        lt = vw < bv
        return jnp.where(lt, vw, bv), jnp.where(lt, cw, bc)

    init = (jnp.full((16,), jnp.inf, jnp.float32), jnp.zeros((16,), jnp.int32))
    carry = lax.fori_loop(0, _HITERS, body, init)
    cp_p1.wait()
    cp_u1.wait()
    cp_pt.wait()
    cp_ut.wait()
    bv, bc = lax.fori_loop(_HITERS, _ITERS_T, body, carry)
    in_tail = bc >= _ITERS * _UNROLL
    base = jnp.where(in_tail, _TAILB - _ITERS * _UNROLL * 16, b)
    rv[...] = bv
    ri[...] = base + bc * 16 + lax.iota(jnp.int32, 16)
    o = pl.multiple_of(w * 16, 16)
    pltpu.sync_copy(rv, vals_out.at[pl.ds(o, 16)])
    pltpu.sync_copy(ri, idx_out.at[pl.ds(o, 16)])


_TCFULL = (_TAILB - _NSC) // _BLK
_TCPAIRS = _TCFULL // 2
_TAIL = _TAILB - _NSC - _TCFULL * _BLK


def _tc_chunk_min(v, base, rows):
    gidx = (lax.broadcasted_iota(jnp.int32, (rows, 128), 0) * 128
            + lax.broadcasted_iota(jnp.int32, (rows, 128), 1) + base)
    m = jnp.min(v)
    ci = jnp.min(jnp.where(v == m, gidx, jnp.int32(2**31 - 1)))
    return m, ci


def _tc_scan_body(p_hbm, u_hbm, val_out, idx_out, pb0, ub0, pb1, ub1, sems):
    def start(g, pb, ub, si):
        src = pl.ds(pl.multiple_of(_NSC + g * _BLK, 128), _BLK)
        pltpu.make_async_copy(p_hbm.at[src], pb, sems.at[si]).start()
        pltpu.make_async_copy(u_hbm.at[src], ub, sems.at[si + 1]).start()

    def wait(g, pb, ub, si):
        src = pl.ds(pl.multiple_of(_NSC + g * _BLK, 128), _BLK)
        pltpu.make_async_copy(p_hbm.at[src], pb, sems.at[si]).wait()
        pltpu.make_async_copy(u_hbm.at[src], ub, sems.at[si + 1]).wait()

    _SUB = _BLK // 1024

    def chunk(g, pb, ub, carry):
        bv8, bc8 = carry
        pv = pb[...].reshape(_BLK // 128, 128)
        uv = ub[...].reshape(_BLK // 128, 128)
        v = (jnp.float32(0.0) - jnp.log(uv)) * jnp.exp(jnp.float32(0.0) - pv)
        for s in range(_SUB):
            sub = v[s * 8:(s + 1) * 8, :]
            code = g * _SUB + s
            lt = sub < bv8
            bv8 = jnp.where(lt, sub, bv8)
            bc8 = jnp.where(lt, code, bc8)
        return bv8, bc8

    start(0, pb0, ub0, 0)

    def body(i, carry):
        g = i * 2
        wait(g, pb0, ub0, 0)
        start(g + 1, pb1, ub1, 2)
        carry = chunk(g, pb0, ub0, carry)
        wait(g + 1, pb1, ub1, 2)

        @pl.when(g + 2 < _TCFULL)
        def _():
            start(g + 2, pb0, ub0, 0)

        return chunk(g + 1, pb1, ub1, carry)

    init = (jnp.full((8, 128), jnp.inf, jnp.float32),
            jnp.zeros((8, 128), jnp.int32))
    carry = lax.fori_loop(0, _TCPAIRS, body, init)
    if _TCFULL % 2 == 1:
        g = _TCFULL - 1
        wait(g, pb0, ub0, 0)
        carry = chunk(g, pb0, ub0, carry)
    bv8, bc8 = carry

    gidx8 = (_NSC + bc8 * 1024
             + lax.broadcasted_iota(jnp.int32, (8, 128), 0) * 128
             + lax.broadcasted_iota(jnp.int32, (8, 128), 1))
    bv = jnp.min(bv8)
    bi = jnp.min(jnp.where(bv8 == bv, gidx8, jnp.int32(2**31 - 1)))

    tsrc = pl.ds(_NSC + _TCFULL * _BLK, _TAIL)
    pltpu.make_async_copy(p_hbm.at[tsrc], pb0.at[pl.ds(0, _TAIL)], sems.at[0]).start()
    pltpu.make_async_copy(u_hbm.at[tsrc], ub0.at[pl.ds(0, _TAIL)], sems.at[1]).start()
    pltpu.make_async_copy(p_hbm.at[tsrc], pb0.at[pl.ds(0, _TAIL)], sems.at[0]).wait()
    pltpu.make_async_copy(u_hbm.at[tsrc], ub0.at[pl.ds(0, _TAIL)], sems.at[1]).wait()
    pv = pb0[pl.ds(0, _TAIL)].reshape(_TAIL // 128, 128)
    uv = ub0[pl.ds(0, _TAIL)].reshape(_TAIL // 128, 128)
    v = (jnp.float32(0.0) - jnp.log(uv)) * jnp.exp(jnp.float32(0.0) - pv)
    m, ci = _tc_chunk_min(v, _NSC + _TCFULL * _BLK, _TAIL // 128)
    better = (m < bv) | ((m == bv) & (ci < bi))
    val_out[0, 0] = jnp.where(better, m, bv)
    idx_out[0, 0] = jnp.where(better, ci, bi)


_tc_scan = pl.pallas_call(
    _tc_scan_body,
    in_specs=[pl.BlockSpec(memory_space=pl.ANY),
              pl.BlockSpec(memory_space=pl.ANY)],
    out_specs=[pl.BlockSpec(memory_space=pltpu.SMEM),
               pl.BlockSpec(memory_space=pltpu.SMEM)],
    out_shape=[jax.ShapeDtypeStruct((1, 1), jnp.float32),
               jax.ShapeDtypeStruct((1, 1), jnp.int32)],
    scratch_shapes=[pltpu.VMEM((_BLK,), jnp.float32),
                    pltpu.VMEM((_BLK,), jnp.float32),
                    pltpu.VMEM((_BLK,), jnp.float32),
                    pltpu.VMEM((_BLK,), jnp.float32),
                    pltpu.SemaphoreType.DMA((4,))],
)


def _merge_body(v_ref, i_ref, tv_ref, ti_ref, o_ref):
    v = v_ref[...].reshape(4, 128)
    ix = i_ref[...].reshape(4, 128)
    m = jnp.min(v)
    i_sc = jnp.min(jnp.where(v == m, ix, jnp.int32(2**31 - 1)))
    tv = tv_ref[0, 0]
    ti = ti_ref[0, 0]
    tc_wins = (tv < m) | ((tv == m) & (ti < i_sc))
    o_ref[0, 0] = jnp.where(tc_wins, ti, i_sc)


_merge = pl.pallas_call(
    _merge_body,
    in_specs=[pl.BlockSpec(memory_space=pltpu.VMEM),
              pl.BlockSpec(memory_space=pltpu.VMEM),
              pl.BlockSpec(memory_space=pltpu.SMEM),
              pl.BlockSpec(memory_space=pltpu.SMEM)],
    out_specs=pl.BlockSpec(memory_space=pltpu.SMEM),
    out_shape=jax.ShapeDtypeStruct((1, 1), jnp.int32),
)


def kernel(p, u):
    vals, idx = _sc_scan(p, u)
    tv, ti = _tc_scan(p, u)
    out = _merge(vals, idx, tv, ti)
    return out[0, 0]
